# baseline (device time: 376054 ns/iter reference)
import jax
import jax.numpy as jnp
from jax import lax
from jax.experimental import pallas as pl
from jax.experimental.pallas import tpu as pltpu

N_DEV = 8
SUB = 4


def kernel(x, w_mat):
    m, k_shard = x.shape
    _, n = w_mat.shape
    m_blk = m // N_DEV
    half = n // 2
    sw = half // SUB

    x = x.astype(jnp.bfloat16)
    w_mat = w_mat.astype(jnp.bfloat16)

    def body(
        x_hbm,
        w_ref,
        out_ref,
        x_blk,
        send_buf,
        recv_buf,
        x_sems,
        send_sems,
        recv_sems,
        credit_sems,
    ):
        d = lax.axis_index("i")
        right = lax.rem(d + 1, N_DEV)
        left = lax.rem(d + N_DEV - 1, N_DEV)
        dir_dst = (right, left)
        credit_dst = (left, right)

        barrier_sem = pltpu.get_barrier_semaphore()

        def nbr_barrier():
            for nbr in (left, right):
                pl.semaphore_signal(
                    barrier_sem,
                    inc=1,
                    device_id=(nbr,),
                    device_id_type=pl.DeviceIdType.MESH,
                )
            pl.semaphore_wait(barrier_sem, 2)

        def rdma(dirn, c, parity):
            col = pl.ds(dirn * half + c * sw, sw)
            return pltpu.make_async_remote_copy(
                src_ref=send_buf.at[:, col],
                dst_ref=recv_buf.at[parity, :, col],
                send_sem=send_sems.at[dirn, c, parity],
                recv_sem=recv_sems.at[dirn, c, parity],
                device_id=(dir_dst[dirn],),
                device_id_type=pl.DeviceIdType.MESH,
            )

        for s in range(N_DEV):
            t_cw = lax.rem(d + (N_DEV - 1 - s), N_DEV)
            t_ccw = lax.rem(d + s + 1, N_DEV)
            for dirn, t in ((0, t_cw), (1, t_ccw)):
                pltpu.make_async_copy(
                    x_hbm.at[pl.ds(t * m_blk, m_blk), :],
                    x_blk.at[dirn],
                    x_sems.at[dirn],
                ).start()

            if s == 0:
                nbr_barrier()

            for dirn in (0, 1):
                pltpu.make_async_copy(
                    x_hbm.at[pl.ds((t_cw if dirn == 0 else t_ccw) * m_blk, m_blk), :],
                    x_blk.at[dirn],
                    x_sems.at[dirn],
                ).wait()

            def sub_body(c, _, s=s):
                for dirn in (0, 1):
                    col = pl.ds(dirn * half + c * sw, sw)
                    if s >= 1:
                        rdma(dirn, c, (s - 1) % 2).wait_send()
                        rdma(dirn, c, (s - 1) % 2).wait_recv()
                    contrib = jnp.dot(
                        x_blk[dirn],
                        w_ref[:, col],
                        preferred_element_type=jnp.float32,
                    )
                    if s >= 1:
                        contrib = contrib + recv_buf[
                            (s - 1) % 2, :, col
                        ].astype(jnp.float32)
                    if s < N_DEV - 1:
                        send_buf[:, col] = contrib.astype(jnp.bfloat16)
                        if 1 <= s <= 5:
                            pl.semaphore_signal(
                                credit_sems.at[dirn, c],
                                inc=1,
                                device_id=(credit_dst[dirn],),
                                device_id_type=pl.DeviceIdType.MESH,
                            )
                        if 2 <= s <= 6:
                            pl.semaphore_wait(credit_sems.at[dirn, c], 1)
                        rdma(dirn, c, s % 2).start()
                    else:
                        out_ref[:, col] = contrib
                return 0

            lax.fori_loop(0, SUB, sub_body, 0)

    return pl.pallas_call(
        body,
        out_shape=jax.ShapeDtypeStruct((m_blk, n), jnp.float32),
        in_specs=[
            pl.BlockSpec(memory_space=pl.ANY),
            pl.BlockSpec(memory_space=pltpu.VMEM),
        ],
        out_specs=pl.BlockSpec(memory_space=pltpu.VMEM),
        scratch_shapes=[
            pltpu.VMEM((2, m_blk, k_shard), jnp.bfloat16),
            pltpu.VMEM((m_blk, n), jnp.bfloat16),
            pltpu.VMEM((2, m_blk, n), jnp.bfloat16),
            pltpu.SemaphoreType.DMA((2,)),
            pltpu.SemaphoreType.DMA((2, SUB, 2)),
            pltpu.SemaphoreType.DMA((2, SUB, 2)),
            pltpu.SemaphoreType.REGULAR((2, SUB)),
        ],
        compiler_params=pltpu.CompilerParams(
            collective_id=0,
            vmem_limit_bytes=60 * 1024 * 1024,
        ),
    )(x, w_mat)


# device time: 360963 ns/iter; 1.0418x vs baseline; 1.0418x over previous
import jax
import jax.numpy as jnp
from jax import lax
from jax.experimental import pallas as pl
from jax.experimental.pallas import tpu as pltpu

N_DEV = 8
SUB = 4


def kernel(x, w_mat):
    m, k_shard = x.shape
    _, n = w_mat.shape
    m_blk = m // N_DEV
    half = n // 2
    sw = half // SUB

    w_mat = w_mat.astype(jnp.bfloat16)

    def body(
        x_hbm,
        w_ref,
        out_ref,
        x_f32,
        x_blk,
        send_buf,
        recv_buf,
        x_sems,
        send_sems,
        recv_sems,
    ):
        d = lax.axis_index("i")
        right = lax.rem(d + 1, N_DEV)
        left = lax.rem(d + N_DEV - 1, N_DEV)
        dir_dst = (right, left)

        barrier_sem = pltpu.get_barrier_semaphore()

        def nbr_barrier():
            for nbr in (left, right):
                pl.semaphore_signal(
                    barrier_sem,
                    inc=1,
                    device_id=(nbr,),
                    device_id_type=pl.DeviceIdType.MESH,
                )
            pl.semaphore_wait(barrier_sem, 2)

        def rdma(dirn, c, parity):
            col = pl.ds(dirn * half + c * sw, sw)
            return pltpu.make_async_remote_copy(
                src_ref=send_buf.at[:, col],
                dst_ref=recv_buf.at[parity, :, col],
                send_sem=send_sems.at[dirn, c, parity],
                recv_sem=recv_sems.at[dirn, c, parity],
                device_id=(dir_dst[dirn],),
                device_id_type=pl.DeviceIdType.MESH,
            )

        for s in range(N_DEV):
            t_cw = lax.rem(d + (N_DEV - 1 - s), N_DEV)
            t_ccw = lax.rem(d + s + 1, N_DEV)

            nbr_barrier()

            for dirn, t in ((0, t_cw), (1, t_ccw)):
                fetch = pltpu.make_async_copy(
                    x_hbm.at[pl.ds(t * m_blk, m_blk), :],
                    x_f32,
                    x_sems.at[dirn],
                )
                fetch.start()
                fetch.wait()
                x_blk[dirn, :, :] = x_f32[:, :].astype(jnp.bfloat16)

            def sub_body(c, _, s=s):
                for dirn in (0, 1):
                    col = pl.ds(dirn * half + c * sw, sw)
                    if s >= 1:
                        rdma(dirn, c, (s - 1) % 2).wait_send()
                        rdma(dirn, c, (s - 1) % 2).wait_recv()
                    contrib = jnp.dot(
                        x_blk[dirn],
                        w_ref[:, col],
                        preferred_element_type=jnp.float32,
                    )
                    if s >= 1:
                        contrib = contrib + recv_buf[
                            (s - 1) % 2, :, col
                        ].astype(jnp.float32)
                    if s < N_DEV - 1:
                        send_buf[:, col] = contrib.astype(jnp.bfloat16)
                        rdma(dirn, c, s % 2).start()
                    else:
                        out_ref[:, col] = contrib
                return 0

            lax.fori_loop(0, SUB, sub_body, 0)

    return pl.pallas_call(
        body,
        out_shape=jax.ShapeDtypeStruct((m_blk, n), jnp.float32),
        in_specs=[
            pl.BlockSpec(memory_space=pl.ANY),
            pl.BlockSpec(memory_space=pltpu.VMEM),
        ],
        out_specs=pl.BlockSpec(memory_space=pltpu.VMEM),
        scratch_shapes=[
            pltpu.VMEM((m_blk, k_shard), jnp.float32),
            pltpu.VMEM((2, m_blk, k_shard), jnp.bfloat16),
            pltpu.VMEM((m_blk, n), jnp.bfloat16),
            pltpu.VMEM((2, m_blk, n), jnp.bfloat16),
            pltpu.SemaphoreType.DMA((2,)),
            pltpu.SemaphoreType.DMA((2, SUB, 2)),
            pltpu.SemaphoreType.DMA((2, SUB, 2)),
        ],
        compiler_params=pltpu.CompilerParams(
            collective_id=0,
            vmem_limit_bytes=62 * 1024 * 1024,
        ),
    )(x, w_mat)
